# baseline (speedup 1.0000x reference)
import jax
import jax.numpy as jnp
from jax.experimental import pallas as pl
from jax.experimental.pallas import tpu as pltpu

_KBLK = 400


def _bf(a):
    return a.astype(jnp.bfloat16)


def _make_kernel(nblk):
    def _fused(x_ref, wl1_ref, bl1_ref, wgc1_ref, adj_ref, adj2_ref,
               bgc1_ref, wgc2_ref, bgc2_ref, wgc3_ref, bgc3_ref,
               wl4_ref, bl4_ref, out_ref, h_ref):
        i = pl.program_id(0)

        h_blk = jnp.dot(_bf(x_ref[...]), _bf(wl1_ref[...]),
                        preferred_element_type=jnp.float32) + bl1_ref[...]
        h_ref[pl.ds(i * _KBLK, _KBLK), :] = _bf(h_blk)

        @pl.when(i == nblk - 1)
        def _tail():
            t = jax.lax.dot_general(
                h_ref[...], wgc1_ref[...], (((0,), (0,)), ((), ())),
                preferred_element_type=jnp.float32)
            adj_bf = adj_ref[...]
            h1 = jnp.maximum(
                jnp.dot(adj_bf, _bf(t), preferred_element_type=jnp.float32)
                + bgc1_ref[...], 0.0)
            s2 = jnp.dot(_bf(h1), _bf(wgc2_ref[...]),
                         preferred_element_type=jnp.float32)
            h2 = jnp.dot(adj_bf, _bf(s2),
                         preferred_element_type=jnp.float32) + bgc2_ref[...]
            s3 = jnp.dot(_bf(h2), _bf(wgc3_ref[...]),
                         preferred_element_type=jnp.float32)
            h3 = jnp.dot(adj2_ref[...], _bf(s3),
                         preferred_element_type=jnp.float32) + bgc3_ref[...]
            w = jnp.dot(_bf(h3), _bf(wl4_ref[...]),
                        preferred_element_type=jnp.float32) + bl4_ref[...]
            m = jnp.max(w, axis=1, keepdims=True)
            shifted = w - m
            out_ref[...] = shifted - jnp.log(
                jnp.sum(jnp.exp(shifted), axis=1, keepdims=True))

    return _fused


def kernel(x, adj, adj2, W_l1, b_l1, W_gc1, b_gc1, W_gc2, b_gc2,
           W_gc3, b_gc3, W_l4, b_l4):
    nfeat, nin = x.shape
    nhid = W_gc1.shape[1]
    n = adj.shape[0]
    nout = W_l4.shape[1]
    nblk = nfeat // _KBLK

    adj_bf = adj.astype(jnp.bfloat16)
    adj2_bf = adj2.astype(jnp.bfloat16)
    wgc1_bf = W_gc1.astype(jnp.bfloat16)

    full = lambda a: pl.BlockSpec(a.shape, lambda i: (0,) * a.ndim)
    out = pl.pallas_call(
        _make_kernel(nblk),
        grid=(nblk,),
        in_specs=[
            pl.BlockSpec((_KBLK, nin), lambda i: (i, 0)),
            full(W_l1),
            pl.BlockSpec((1, n), lambda i: (0, 0)),
            full(wgc1_bf),
            full(adj_bf),
            full(adj2_bf),
            pl.BlockSpec((1, nhid), lambda i: (0, 0)),
            full(W_gc2),
            pl.BlockSpec((1, W_gc2.shape[1]), lambda i: (0, 0)),
            full(W_gc3),
            pl.BlockSpec((1, W_gc3.shape[1]), lambda i: (0, 0)),
            full(W_l4),
            pl.BlockSpec((1, nout), lambda i: (0, 0)),
        ],
        out_specs=pl.BlockSpec((n, nout), lambda i: (0, 0)),
        out_shape=jax.ShapeDtypeStruct((n, nout), jnp.float32),
        scratch_shapes=[
            pltpu.VMEM((nfeat, n), jnp.bfloat16),
        ],
    )(x, W_l1, b_l1.reshape(1, n), wgc1_bf, adj_bf, adj2_bf,
      b_gc1.reshape(1, -1), W_gc2, b_gc2.reshape(1, -1),
      W_gc3, b_gc3.reshape(1, -1), W_l4, b_l4.reshape(1, -1))
    return out
